# Initial kernel scaffold; baseline (speedup 1.0000x reference)
import functools

import jax
import jax.numpy as jnp
from jax import lax
from jax.experimental import pallas as pl
from jax.experimental.pallas import tpu as pltpu
from jax.experimental.pallas import tpu_sc as plsc

B = 16384
H = 200
D = 32
NC = 2
NS = 16
NW = NC * NS
RPT = B // NW
CB = 8
NCHUNK = RPT // CB
GLEN = 100
GPC = CB * H // GLEN

_mesh = plsc.VectorSubcoreMesh(core_axis_name="c", subcore_axis_name="s")


@functools.partial(
    pl.kernel,
    out_type=jax.ShapeDtypeStruct((B, D), jnp.float32),
    mesh=_mesh,
    scratch_types=[
        pltpu.VMEM((GPC, GLEN), jnp.int32),
        pltpu.VMEM((CB * H, D), jnp.float32),
        pltpu.VMEM((CB, D), jnp.float32),
        pltpu.SemaphoreType.DMA,
    ],
)
def _pool(idx_hbm, table_hbm, out_hbm, idx_v, rows_v, out_v, sem):
    wid = lax.axis_index("s") * NC + lax.axis_index("c")
    tile_base = wid * RPT

    def chunk_body(ci, carry):
        row0 = tile_base + ci * CB
        pltpu.sync_copy(idx_hbm.at[pl.ds(row0 * (H // GLEN), GPC)], idx_v)
        copies = [
            pltpu.async_copy(
                table_hbm.at[idx_v.at[g]],
                rows_v.at[pl.ds(g * GLEN, GLEN)],
                sem,
            )
            for g in range(GPC)
        ]
        for c in copies:
            c.wait()

        inv = jnp.float32(1.0 / H)
        for b in range(CB):
            def h_body(h, accs, _b=b):
                a0, a1 = accs
                r = _b * H + h
                return (a0 + rows_v[r, pl.ds(0, 16)],
                        a1 + rows_v[r, pl.ds(16, 16)])

            zero = jnp.zeros((16,), jnp.float32)
            a0, a1 = lax.fori_loop(0, H, h_body, (zero, zero))
            out_v[b, pl.ds(0, 16)] = a0 * inv
            out_v[b, pl.ds(16, 16)] = a1 * inv

        pltpu.sync_copy(out_v, out_hbm.at[pl.ds(row0, CB)])
        return carry

    lax.fori_loop(0, NCHUNK, chunk_body, 0)


def kernel(item_tensors, table):
    idx2 = item_tensors.reshape(B * (H // GLEN), GLEN)
    return _pool(idx2, table)


# speedup vs baseline: 10.6007x; 10.6007x over previous
import functools

import jax
import jax.numpy as jnp
from jax import lax
from jax.experimental import pallas as pl
from jax.experimental.pallas import tpu as pltpu
from jax.experimental.pallas import tpu_sc as plsc

B = 16384
H = 200
D = 32
NC = 2
NS = 16
NW = NC * NS
RPT = B // NW
CB = 8
NCHUNK = RPT // CB
GLEN = 100
GPC = CB * H // GLEN

_mesh = plsc.VectorSubcoreMesh(core_axis_name="c", subcore_axis_name="s")


@functools.partial(
    pl.kernel,
    out_type=jax.ShapeDtypeStruct((B, D), jnp.float32),
    mesh=_mesh,
    scratch_types=[
        pltpu.VMEM((GPC, GLEN), jnp.int32),
        pltpu.VMEM((CB * H, D), jnp.float32),
        pltpu.VMEM((CB, D), jnp.float32),
        pltpu.SemaphoreType.DMA,
    ],
    compiler_params=pltpu.CompilerParams(use_tc_tiling_on_sc=False),
)
def _pool(idx_hbm, table_hbm, out_hbm, idx_v, rows_v, out_v, sem):
    wid = lax.axis_index("s") * NC + lax.axis_index("c")
    tile_base = wid * RPT

    def chunk_body(ci, carry):
        row0 = tile_base + ci * CB
        pltpu.sync_copy(idx_hbm.at[pl.ds(row0 * (H // GLEN), GPC)], idx_v)
        copies = [
            pltpu.async_copy(
                table_hbm.at[idx_v.at[g]],
                rows_v.at[pl.ds(g * GLEN, GLEN)],
                sem,
            )
            for g in range(GPC)
        ]
        for c in copies:
            c.wait()

        inv = jnp.float32(1.0 / H)
        for b in range(CB):
            def h_body(h, accs, _b=b):
                a0, a1 = accs
                r = _b * H + h
                return (a0 + rows_v[r, pl.ds(0, 16)],
                        a1 + rows_v[r, pl.ds(16, 16)])

            zero = jnp.zeros((16,), jnp.float32)
            a0, a1 = lax.fori_loop(0, H, h_body, (zero, zero))
            out_v[b, pl.ds(0, 16)] = a0 * inv
            out_v[b, pl.ds(16, 16)] = a1 * inv

        pltpu.sync_copy(out_v, out_hbm.at[pl.ds(row0, CB)])
        return carry

    lax.fori_loop(0, NCHUNK, chunk_body, 0)


def kernel(item_tensors, table):
    idx2 = item_tensors.reshape(B * (H // GLEN), GLEN)
    return _pool(idx2, table)


# speedup vs baseline: 16.2402x; 1.5320x over previous
import functools

import jax
import jax.numpy as jnp
from jax import lax
from jax.experimental import pallas as pl
from jax.experimental.pallas import tpu as pltpu
from jax.experimental.pallas import tpu_sc as plsc

B = 16384
H = 200
D = 32
NC = 2
NS = 16
NW = NC * NS
RPT = B // NW
CB = 8
NCHUNK = RPT // CB
GLEN = 100
GPC = CB * H // GLEN
U = 4

_mesh = plsc.VectorSubcoreMesh(core_axis_name="c", subcore_axis_name="s")


@functools.partial(
    pl.kernel,
    out_type=jax.ShapeDtypeStruct((B, D), jnp.float32),
    mesh=_mesh,
    scratch_types=[
        pltpu.VMEM((2, GPC, GLEN), jnp.int32),
        pltpu.VMEM((2, CB * H, D), jnp.float32),
        pltpu.VMEM((CB, D), jnp.float32),
        pltpu.SemaphoreType.DMA,
        pltpu.SemaphoreType.DMA,
        pltpu.SemaphoreType.DMA,
        pltpu.SemaphoreType.DMA,
    ],
    compiler_params=pltpu.CompilerParams(use_tc_tiling_on_sc=False),
)
def _pool(idx_hbm, table_hbm, out_hbm, idx_v, rows_v, out_v,
          gsem0, gsem1, isem0, isem1):
    gsems = (gsem0, gsem1)
    isems = (isem0, isem1)
    wid = lax.axis_index("s") * NC + lax.axis_index("c")
    tile_base = wid * RPT

    def fire_idx(ci, buf):
        row0 = tile_base + ci * CB
        pltpu.async_copy(idx_hbm.at[pl.ds(row0 * (H // GLEN), GPC)],
                         idx_v.at[buf], isems[buf])

    def drain_idx(buf):
        pltpu.make_async_copy(idx_hbm.at[pl.ds(0, GPC)],
                              idx_v.at[buf], isems[buf]).wait()

    def fire_gathers(buf):
        for g in range(GPC):
            pltpu.async_copy(table_hbm.at[idx_v.at[buf, g]],
                             rows_v.at[buf, pl.ds(g * GLEN, GLEN)],
                             gsems[buf])

    def drain_gathers(buf):
        pltpu.make_async_copy(table_hbm.at[pl.ds(0, CB * H)],
                              rows_v.at[buf], gsems[buf]).wait()

    def accumulate_store(ci, buf):
        row0 = tile_base + ci * CB
        inv = jnp.float32(1.0 / H)
        for b in range(CB):
            def h_body(k, accs, _b=b, _buf=buf):
                r = _b * H + k * U
                nxt = []
                for u in range(U):
                    nxt.append(accs[2 * u] + rows_v[_buf, r + u, pl.ds(0, 16)])
                    nxt.append(accs[2 * u + 1] + rows_v[_buf, r + u, pl.ds(16, 16)])
                return tuple(nxt)

            zero = jnp.zeros((16,), jnp.float32)
            accs = lax.fori_loop(0, H // U, h_body, (zero,) * (2 * U))
            a0 = (accs[0] + accs[2]) + (accs[4] + accs[6])
            a1 = (accs[1] + accs[3]) + (accs[5] + accs[7])
            out_v[b, pl.ds(0, 16)] = a0 * inv
            out_v[b, pl.ds(16, 16)] = a1 * inv
        pltpu.sync_copy(out_v, out_hbm.at[pl.ds(row0, CB)])

    fire_idx(0, 0)
    drain_idx(0)
    fire_gathers(0)
    fire_idx(1, 1)

    @pl.loop(0, NCHUNK, step=2)
    def _outer(ci0):
        for par in range(2):
            ci = ci0 + par
            buf, nbuf = par, 1 - par

            @pl.when(ci + 1 < NCHUNK)
            def _():
                drain_idx(nbuf)

            drain_gathers(buf)

            @pl.when(ci + 1 < NCHUNK)
            def _():
                fire_gathers(nbuf)

            @pl.when(ci + 2 < NCHUNK)
            def _():
                fire_idx(ci + 2, buf)

            accumulate_store(ci, buf)


def kernel(item_tensors, table):
    idx2 = item_tensors.reshape(B * (H // GLEN), GLEN)
    return _pool(idx2, table)


# speedup vs baseline: 16.2510x; 1.0007x over previous
import functools

import jax
import jax.numpy as jnp
from jax import lax
from jax.experimental import pallas as pl
from jax.experimental.pallas import tpu as pltpu
from jax.experimental.pallas import tpu_sc as plsc

B = 16384
H = 200
D = 32
NC = 2
NS = 16
NW = NC * NS
RPT = B // NW
CB = 8
NCHUNK = RPT // CB
GLEN = 100
GPC = CB * H // GLEN
U = 8

_mesh = plsc.VectorSubcoreMesh(core_axis_name="c", subcore_axis_name="s")


@functools.partial(
    pl.kernel,
    out_type=jax.ShapeDtypeStruct((B, D), jnp.float32),
    mesh=_mesh,
    scratch_types=[
        pltpu.VMEM((2, GPC, GLEN), jnp.int32),
        pltpu.VMEM((2, CB * H, D), jnp.float32),
        pltpu.VMEM((2, CB, D), jnp.float32),
        pltpu.SemaphoreType.DMA,
        pltpu.SemaphoreType.DMA,
        pltpu.SemaphoreType.DMA,
        pltpu.SemaphoreType.DMA,
        pltpu.SemaphoreType.DMA,
        pltpu.SemaphoreType.DMA,
    ],
    compiler_params=pltpu.CompilerParams(use_tc_tiling_on_sc=False),
)
def _pool(idx_hbm, table_hbm, out_hbm, idx_v, rows_v, out_v,
          gsem0, gsem1, isem0, isem1, osem0, osem1):
    gsems = (gsem0, gsem1)
    isems = (isem0, isem1)
    osems = (osem0, osem1)
    wid = lax.axis_index("s") * NC + lax.axis_index("c")
    tile_base = wid * RPT

    def fire_idx(ci, buf):
        row0 = tile_base + ci * CB
        pltpu.async_copy(idx_hbm.at[pl.ds(row0 * (H // GLEN), GPC)],
                         idx_v.at[buf], isems[buf])

    def drain_idx(buf):
        pltpu.make_async_copy(idx_hbm.at[pl.ds(0, GPC)],
                              idx_v.at[buf], isems[buf]).wait()

    def fire_gathers(buf):
        for g in range(GPC):
            pltpu.async_copy(table_hbm.at[idx_v.at[buf, g]],
                             rows_v.at[buf, pl.ds(g * GLEN, GLEN)],
                             gsems[buf])

    def drain_gathers(buf):
        pltpu.make_async_copy(table_hbm.at[pl.ds(0, CB * H)],
                              rows_v.at[buf], gsems[buf]).wait()

    def accumulate_store(ci, buf):
        row0 = tile_base + ci * CB
        inv = jnp.float32(1.0 / H)
        for b in range(CB):
            def h_body(k, accs, _b=b, _buf=buf):
                r = _b * H + k * U
                nxt = []
                for u in range(U):
                    nxt.append(accs[2 * u] + rows_v[_buf, r + u, pl.ds(0, 16)])
                    nxt.append(accs[2 * u + 1] + rows_v[_buf, r + u, pl.ds(16, 16)])
                return tuple(nxt)

            zero = jnp.zeros((16,), jnp.float32)
            accs = lax.fori_loop(0, H // U, h_body, (zero,) * (2 * U))
            a0 = accs[0]
            a1 = accs[1]
            for u in range(1, U):
                a0 = a0 + accs[2 * u]
                a1 = a1 + accs[2 * u + 1]
            out_v[buf, b, pl.ds(0, 16)] = a0 * inv
            out_v[buf, b, pl.ds(16, 16)] = a1 * inv
        pltpu.async_copy(out_v.at[buf], out_hbm.at[pl.ds(row0, CB)],
                         osems[buf])

    def drain_out(buf):
        pltpu.make_async_copy(out_v.at[buf], out_hbm.at[pl.ds(0, CB)],
                              osems[buf]).wait()

    fire_idx(0, 0)
    drain_idx(0)
    fire_gathers(0)
    fire_idx(1, 1)

    @pl.loop(0, NCHUNK, step=2)
    def _outer(ci0):
        for par in range(2):
            ci = ci0 + par
            buf, nbuf = par, 1 - par

            @pl.when(ci + 1 < NCHUNK)
            def _():
                drain_idx(nbuf)

            drain_gathers(buf)

            @pl.when(ci + 1 < NCHUNK)
            def _():
                fire_gathers(nbuf)

            @pl.when(ci + 2 < NCHUNK)
            def _():
                fire_idx(ci + 2, buf)

            @pl.when(ci >= 2)
            def _():
                drain_out(buf)

            accumulate_store(ci, buf)

    drain_out(0)
    drain_out(1)


def kernel(item_tensors, table):
    idx2 = item_tensors.reshape(B * (H // GLEN), GLEN)
    return _pool(idx2, table)
